# baseline (speedup 1.0000x reference)
import functools

import jax
import jax.numpy as jnp
from jax import lax
from jax.experimental import pallas as pl
from jax.experimental.pallas import tpu as pltpu
from jax.experimental.pallas import tpu_sc as plsc

B = 16384
D = 64


def _widen_body(u_ref, m_ref, ou_ref, om_ref):
    def widen(x):
        shifted = jnp.concatenate([x[1:, :], x[:1, :]], axis=0)
        return jnp.concatenate([x, shifted], axis=1)

    ou_ref[...] = widen(u_ref[...])
    om_ref[...] = widen(m_ref[...])


def _widen(user_emb, movie_emb):
    n = user_emb.shape[0]
    rblk = 1000
    grid = n // rblk
    row = lambda i: (i, 0)
    return pl.pallas_call(
        _widen_body,
        grid=(grid,),
        in_specs=[
            pl.BlockSpec((rblk, D), row),
            pl.BlockSpec((rblk, D), row),
        ],
        out_specs=[
            pl.BlockSpec((rblk, 2 * D), row),
            pl.BlockSpec((rblk, 2 * D), row),
        ],
        out_shape=[
            jax.ShapeDtypeStruct((n, 2 * D), jnp.float32),
            jax.ShapeDtypeStruct((n, 2 * D), jnp.float32),
        ],
    )(user_emb, movie_emb)


def _sc_gather(uid, m1id, m2id, user_wide, movie_wide):
    info = plsc.get_sparse_core_info()
    nc, ns = info.num_cores, info.num_subcores
    nw = nc * ns
    bpw = B // nw
    nchunk = bpw // 128

    mesh = plsc.VectorSubcoreMesh(core_axis_name="c", subcore_axis_name="s")

    @functools.partial(
        pl.kernel,
        mesh=mesh,
        out_type=[jax.ShapeDtypeStruct((B, 2 * D), jnp.float32)] * 3,
        scratch_types=[
            pltpu.VMEM((nchunk, 128), jnp.int32),
            pltpu.VMEM((nchunk, 128), jnp.int32),
            pltpu.VMEM((nchunk, 128), jnp.int32),
            pltpu.VMEM((2, 128, 2 * D), jnp.float32),
            pltpu.VMEM((2, 128, 2 * D), jnp.float32),
            pltpu.VMEM((2, 128, 2 * D), jnp.float32),
            pltpu.SemaphoreType.DMA,
        ],
    )
    def gather_k(uid_hbm, m1_hbm, m2_hbm, uemb_hbm, memb_hbm,
                 out_u, out_1, out_2,
                 idx_u, idx_1, idx_2, rows_u, rows_1, rows_2, sem):
        wid = lax.axis_index("s") * nc + lax.axis_index("c")
        base = wid * bpw
        pltpu.sync_copy(uid_hbm.at[pl.ds(wid * nchunk, nchunk)], idx_u)
        pltpu.sync_copy(m1_hbm.at[pl.ds(wid * nchunk, nchunk)], idx_1)
        pltpu.sync_copy(m2_hbm.at[pl.ds(wid * nchunk, nchunk)], idx_2)
        pend = None
        for j in range(nchunk + 1):
            if j < nchunk:
                buf = j % 2
                pend_next = [
                    pltpu.async_copy(uemb_hbm.at[idx_u.at[j]],
                                     rows_u.at[buf], sem),
                    pltpu.async_copy(memb_hbm.at[idx_1.at[j]],
                                     rows_1.at[buf], sem),
                    pltpu.async_copy(memb_hbm.at[idx_2.at[j]],
                                     rows_2.at[buf], sem),
                ]
            if j >= 1:
                for c in pend:
                    c.wait()
                buf = (j - 1) % 2
                dst = pl.ds(base + (j - 1) * 128, 128)
                pltpu.sync_copy(rows_u.at[buf], out_u.at[dst])
                pltpu.sync_copy(rows_1.at[buf], out_1.at[dst])
                pltpu.sync_copy(rows_2.at[buf], out_2.at[dst])
            if j < nchunk:
                pend = pend_next

    return gather_k(uid, m1id, m2id, user_wide, movie_wide)


def _mlp_body(u_ref, v1_ref, v2_ref, w1u_ref, w1m_ref, b1_ref, w2_ref,
              b2_ref, w3_ref, o_ref):
    u = u_ref[:, :D]
    uw = jnp.dot(u, w1u_ref[...], preferred_element_type=jnp.float32)

    def head(v_ref):
        h = uw + jnp.dot(v_ref[:, :D], w1m_ref[...],
                         preferred_element_type=jnp.float32) + b1_ref[...]
        h = jnp.maximum(h, 0.0)
        h = jnp.dot(h, w2_ref[...], preferred_element_type=jnp.float32)
        h = jnp.maximum(h + b2_ref[...], 0.0)
        return h

    dh = head(v1_ref) - head(v2_ref)
    o_ref[...] = jnp.sum(dh * w3_ref[...], axis=1, keepdims=True)


def _tc_mlp(u, v1, v2, w1u, w1m, b1, w2, b2, w3):
    blk = 2048
    grid = B // blk
    row = lambda i: (i, 0)
    const = lambda i: (0, 0)
    return pl.pallas_call(
        _mlp_body,
        grid=(grid,),
        in_specs=[
            pl.BlockSpec((blk, 2 * D), row),
            pl.BlockSpec((blk, 2 * D), row),
            pl.BlockSpec((blk, 2 * D), row),
            pl.BlockSpec((D, 16), const),
            pl.BlockSpec((D, 16), const),
            pl.BlockSpec((1, 16), const),
            pl.BlockSpec((16, 8), const),
            pl.BlockSpec((1, 8), const),
            pl.BlockSpec((1, 8), const),
        ],
        out_specs=pl.BlockSpec((blk, 1), row),
        out_shape=jax.ShapeDtypeStruct((B, 1), jnp.float32),
    )(u, v1, v2, w1u, w1m, b1, w2, b2, w3)


def kernel(inputs, user_emb, movie_emb, W1, b1, W2, b2, W3, b3):
    idx = inputs.astype(jnp.int32)
    uid = idx[:, 0].reshape(B // 128, 128)
    m1id = idx[:, 1].reshape(B // 128, 128)
    m2id = idx[:, 2].reshape(B // 128, 128)
    user_wide, movie_wide = _widen(user_emb, movie_emb)
    u, v1, v2 = _sc_gather(uid, m1id, m2id, user_wide, movie_wide)
    return _tc_mlp(
        u, v1, v2,
        W1[:D], W1[D:],
        b1.reshape(1, 16),
        W2,
        b2.reshape(1, 8),
        W3.reshape(1, 8),
    )


# speedup vs baseline: 2.4912x; 2.4912x over previous
import functools

import jax
import jax.numpy as jnp
from jax import lax
from jax.experimental import pallas as pl
from jax.experimental.pallas import tpu as pltpu
from jax.experimental.pallas import tpu_sc as plsc

B = 16384
D = 64


def _sc_gather(uid, m1id, m2id, user_emb, movie_emb):
    info = plsc.get_sparse_core_info()
    nc, ns = info.num_cores, info.num_subcores
    nw = nc * ns
    bpw = B // nw
    nchunk = bpw // 128

    mesh = plsc.VectorSubcoreMesh(core_axis_name="c", subcore_axis_name="s")

    @functools.partial(
        pl.kernel,
        mesh=mesh,
        compiler_params=pltpu.CompilerParams(needs_layout_passes=False),
        out_type=[jax.ShapeDtypeStruct((B, D), jnp.float32)] * 3,
        scratch_types=[
            pltpu.VMEM((nchunk, 128), jnp.int32),
            pltpu.VMEM((nchunk, 128), jnp.int32),
            pltpu.VMEM((nchunk, 128), jnp.int32),
            pltpu.VMEM((bpw // 2, D), jnp.float32),
            pltpu.VMEM((bpw // 2, D), jnp.float32),
            pltpu.VMEM((bpw // 2, D), jnp.float32),
            pltpu.SemaphoreType.DMA,
        ],
    )
    def gather_k(uid_hbm, m1_hbm, m2_hbm, uemb_hbm, memb_hbm,
                 out_u, out_1, out_2,
                 idx_u, idx_1, idx_2, rows_u, rows_1, rows_2, sem):
        wid = lax.axis_index("s") * nc + lax.axis_index("c")
        base = wid * bpw
        half = bpw // 2
        pltpu.sync_copy(uid_hbm.at[pl.ds(wid * nchunk, nchunk)], idx_u)
        pltpu.sync_copy(m1_hbm.at[pl.ds(wid * nchunk, nchunk)], idx_1)
        pltpu.sync_copy(m2_hbm.at[pl.ds(wid * nchunk, nchunk)], idx_2)
        iota = lax.broadcasted_iota(jnp.int32, (16,), 0)
        tables = ((idx_u, uemb_hbm, rows_u, out_u),
                  (idx_1, memb_hbm, rows_1, out_1),
                  (idx_2, memb_hbm, rows_2, out_2))

        for h in range(2):
            def body(q, carry):
                col = q * 16
                for jl in range(nchunk // 2):
                    j = h * (nchunk // 2) + jl
                    for idx_v, emb, rows, _ in tables:
                        vec = idx_v[j, pl.ds(col, 16)]
                        for l in range(16):
                            s = jnp.max(jnp.where(iota == l, vec, 0))
                            row = jl * 128 + col + l
                            pltpu.async_copy(
                                emb.at[pl.ds(s, 1)], rows.at[pl.ds(row, 1)],
                                sem)
                return carry

            lax.fori_loop(0, 8, body, 0)
            for _, emb, rows, _ in tables:
                pltpu.make_async_copy(emb.at[pl.ds(0, half)], rows, sem).wait()
            for _, _, rows, out in tables:
                pltpu.sync_copy(rows, out.at[pl.ds(base + h * half, half)])

    return gather_k(uid, m1id, m2id, user_emb, movie_emb)


def _mlp_body(u_ref, v1_ref, v2_ref, w1u_ref, w1m_ref, b1_ref, w2_ref,
              b2_ref, w3_ref, o_ref):
    uw = jnp.dot(u_ref[...], w1u_ref[...], preferred_element_type=jnp.float32)

    def head(v_ref):
        h = uw + jnp.dot(v_ref[...], w1m_ref[...],
                         preferred_element_type=jnp.float32) + b1_ref[...]
        h = jnp.maximum(h, 0.0)
        h = jnp.dot(h, w2_ref[...], preferred_element_type=jnp.float32)
        h = jnp.maximum(h + b2_ref[...], 0.0)
        return h

    dh = head(v1_ref) - head(v2_ref)
    o_ref[...] = jnp.sum(dh * w3_ref[...], axis=1, keepdims=True)


def _tc_mlp(u, v1, v2, w1u, w1m, b1, w2, b2, w3):
    blk = 2048
    grid = B // blk
    row = lambda i: (i, 0)
    const = lambda i: (0, 0)
    return pl.pallas_call(
        _mlp_body,
        grid=(grid,),
        in_specs=[
            pl.BlockSpec((blk, D), row),
            pl.BlockSpec((blk, D), row),
            pl.BlockSpec((blk, D), row),
            pl.BlockSpec((D, 16), const),
            pl.BlockSpec((D, 16), const),
            pl.BlockSpec((1, 16), const),
            pl.BlockSpec((16, 8), const),
            pl.BlockSpec((1, 8), const),
            pl.BlockSpec((1, 8), const),
        ],
        out_specs=pl.BlockSpec((blk, 1), row),
        out_shape=jax.ShapeDtypeStruct((B, 1), jnp.float32),
    )(u, v1, v2, w1u, w1m, b1, w2, b2, w3)


def kernel(inputs, user_emb, movie_emb, W1, b1, W2, b2, W3, b3):
    idx = inputs.astype(jnp.int32)
    uid = idx[:, 0].reshape(B // 128, 128)
    m1id = idx[:, 1].reshape(B // 128, 128)
    m2id = idx[:, 2].reshape(B // 128, 128)
    u, v1, v2 = _sc_gather(uid, m1id, m2id, user_emb, movie_emb)
    return _tc_mlp(
        u, v1, v2,
        W1[:D], W1[D:],
        b1.reshape(1, 16),
        W2,
        b2.reshape(1, 8),
        W3.reshape(1, 8),
    )


# speedup vs baseline: 2.4916x; 1.0001x over previous
import functools

import jax
import jax.numpy as jnp
from jax import lax
from jax.experimental import pallas as pl
from jax.experimental.pallas import tpu as pltpu
from jax.experimental.pallas import tpu_sc as plsc

B = 16384
D = 64


def _sc_gather(uid, m1id, m2id, user_emb, movie_emb):
    info = plsc.get_sparse_core_info()
    nc, ns = info.num_cores, info.num_subcores
    nw = nc * ns
    bpw = B // nw
    nchunk = bpw // 128

    mesh = plsc.VectorSubcoreMesh(core_axis_name="c", subcore_axis_name="s")

    @functools.partial(
        pl.kernel,
        mesh=mesh,
        compiler_params=pltpu.CompilerParams(needs_layout_passes=False),
        out_type=[jax.ShapeDtypeStruct((B, D), jnp.float32)] * 3,
        scratch_types=[
            pltpu.VMEM((nchunk, 128), jnp.int32),
            pltpu.VMEM((nchunk, 128), jnp.int32),
            pltpu.VMEM((nchunk, 128), jnp.int32),
            pltpu.VMEM((bpw // 2, D), jnp.float32),
            pltpu.VMEM((bpw // 2, D), jnp.float32),
            pltpu.VMEM((bpw // 2, D), jnp.float32),
            pltpu.SemaphoreType.DMA,
            pltpu.SemaphoreType.DMA,
            pltpu.SemaphoreType.DMA,
        ],
    )
    def gather_k(uid_hbm, m1_hbm, m2_hbm, uemb_hbm, memb_hbm,
                 out_u, out_1, out_2,
                 idx_u, idx_1, idx_2, rows_u, rows_1, rows_2,
                 sem_u, sem_1, sem_2):
        wid = lax.axis_index("s") * nc + lax.axis_index("c")
        base = wid * bpw
        half = bpw // 2
        pltpu.sync_copy(uid_hbm.at[pl.ds(wid * nchunk, nchunk)], idx_u)
        pltpu.sync_copy(m1_hbm.at[pl.ds(wid * nchunk, nchunk)], idx_1)
        pltpu.sync_copy(m2_hbm.at[pl.ds(wid * nchunk, nchunk)], idx_2)
        iota = lax.broadcasted_iota(jnp.int32, (16,), 0)
        tables = ((idx_u, uemb_hbm, rows_u, out_u, sem_u),
                  (idx_1, memb_hbm, rows_1, out_1, sem_1),
                  (idx_2, memb_hbm, rows_2, out_2, sem_2))

        for h in range(2):
            def body(q, carry):
                col = q * 16
                for jl in range(nchunk // 2):
                    j = h * (nchunk // 2) + jl
                    for idx_v, emb, rows, _, sem in tables:
                        vec = idx_v[j, pl.ds(col, 16)]
                        for l in range(16):
                            s = jnp.max(jnp.where(iota == l, vec, 0))
                            row = jl * 128 + col + l
                            pltpu.async_copy(
                                emb.at[pl.ds(s, 1)], rows.at[pl.ds(row, 1)],
                                sem)
                return carry

            lax.fori_loop(0, 8, body, 0)
            for _, emb, rows, _, sem in tables:
                pltpu.make_async_copy(emb.at[pl.ds(0, half)], rows, sem).wait()
            for _, _, rows, out, _ in tables:
                pltpu.sync_copy(rows, out.at[pl.ds(base + h * half, half)])

    return gather_k(uid, m1id, m2id, user_emb, movie_emb)


def _mlp_body(u_ref, v1_ref, v2_ref, w1u_ref, w1m_ref, b1_ref, w2_ref,
              b2_ref, w3_ref, o_ref):
    uw = jnp.dot(u_ref[...], w1u_ref[...], preferred_element_type=jnp.float32)

    def head(v_ref):
        h = uw + jnp.dot(v_ref[...], w1m_ref[...],
                         preferred_element_type=jnp.float32) + b1_ref[...]
        h = jnp.maximum(h, 0.0)
        h = jnp.dot(h, w2_ref[...], preferred_element_type=jnp.float32)
        h = jnp.maximum(h + b2_ref[...], 0.0)
        return h

    dh = head(v1_ref) - head(v2_ref)
    o_ref[...] = jnp.sum(dh * w3_ref[...], axis=1, keepdims=True)


def _tc_mlp(u, v1, v2, w1u, w1m, b1, w2, b2, w3):
    blk = 2048
    grid = B // blk
    row = lambda i: (i, 0)
    const = lambda i: (0, 0)
    return pl.pallas_call(
        _mlp_body,
        grid=(grid,),
        in_specs=[
            pl.BlockSpec((blk, D), row),
            pl.BlockSpec((blk, D), row),
            pl.BlockSpec((blk, D), row),
            pl.BlockSpec((D, 16), const),
            pl.BlockSpec((D, 16), const),
            pl.BlockSpec((1, 16), const),
            pl.BlockSpec((16, 8), const),
            pl.BlockSpec((1, 8), const),
            pl.BlockSpec((1, 8), const),
        ],
        out_specs=pl.BlockSpec((blk, 1), row),
        out_shape=jax.ShapeDtypeStruct((B, 1), jnp.float32),
    )(u, v1, v2, w1u, w1m, b1, w2, b2, w3)


def kernel(inputs, user_emb, movie_emb, W1, b1, W2, b2, W3, b3):
    idx = inputs.astype(jnp.int32)
    uid = idx[:, 0].reshape(B // 128, 128)
    m1id = idx[:, 1].reshape(B // 128, 128)
    m2id = idx[:, 2].reshape(B // 128, 128)
    u, v1, v2 = _sc_gather(uid, m1id, m2id, user_emb, movie_emb)
    return _tc_mlp(
        u, v1, v2,
        W1[:D], W1[D:],
        b1.reshape(1, 16),
        W2,
        b2.reshape(1, 8),
        W3.reshape(1, 8),
    )
